# Initial kernel scaffold; baseline (speedup 1.0000x reference)
import jax
import jax.numpy as jnp
from jax.experimental import pallas as pl


def kernel(scaler, vector, batch_index, W1, b1, W2, b2):
    raise NotImplementedError("write your pallas kernel here")


# speedup vs baseline: 4.6427x; 4.6427x over previous
import functools

import jax
import jax.numpy as jnp
from jax import lax
from jax.experimental import pallas as pl
from jax.experimental.pallas import tpu as pltpu
from jax.experimental.pallas import tpu_sc as plsc

NSEG = 1024
NROWS = 320000
D = 128
NW = 32
UNITS = NROWS // 128
BASE_UNITS = UNITS // NW
EXTRA = UNITS - BASE_UNITS * NW


def _sc_segment_sum(scaler, batch_index, zeros):
    mesh = plsc.VectorSubcoreMesh(core_axis_name="c", subcore_axis_name="s")

    @functools.partial(
        pl.kernel,
        mesh=mesh,
        out_type=jax.ShapeDtypeStruct((2 * NSEG, D), jnp.float32),
        scratch_types=[
            pltpu.VMEM_SHARED((NSEG, D), jnp.float32),
            pltpu.VMEM((128, D), jnp.float32),
            pltpu.VMEM((128,), jnp.int32),
            pltpu.VMEM((NSEG // 16, D), jnp.float32),
        ],
    )
    def seg_sum(scaler_hbm, idx_hbm, zeros_hbm, out_hbm, acc, rows, idx, bounce):
        c = lax.axis_index("c")
        s = lax.axis_index("s")
        wid = s * 2 + c

        seg0 = s * (NSEG // 16)
        pltpu.sync_copy(zeros_hbm.at[pl.ds(seg0, NSEG // 16)], bounce)
        pltpu.sync_copy(bounce, acc.at[pl.ds(seg0, NSEG // 16)])
        plsc.subcore_barrier()

        start_u = wid * BASE_UNITS + jnp.minimum(wid, EXTRA)
        nu = BASE_UNITS + jnp.where(wid < EXTRA, 1, 0)

        def body(t, _):
            base = (start_u + t) * 128
            pltpu.sync_copy(idx_hbm.at[pl.ds(base, 128)], idx)
            pltpu.sync_copy(scaler_hbm.at[pl.ds(base, 128)], rows)
            pltpu.sync_copy(rows, acc.at[idx], add=True)
            return ()

        lax.fori_loop(0, nu, body, ())
        plsc.subcore_barrier()

        pltpu.sync_copy(acc.at[pl.ds(seg0, NSEG // 16)], bounce)
        pltpu.sync_copy(bounce, out_hbm.at[pl.ds(c * NSEG + seg0, NSEG // 16)])

    return seg_sum(scaler, batch_index, zeros)


def _mlp_body(p_ref, w1_ref, b1_ref, w2_ref, b2_ref, o_ref):
    x = p_ref[0:NSEG, :] + p_ref[NSEG:2 * NSEG, :]
    h = jnp.dot(x, w1_ref[...], preferred_element_type=jnp.float32) + b1_ref[...]
    h = jnp.maximum(h, 0.0)
    o = jnp.sum(h * w2_ref[...], axis=1, keepdims=True) + b2_ref[0, 0]
    o_ref[...] = o


def _mlp(partials, W1, b1, W2, b2):
    return pl.pallas_call(
        _mlp_body,
        out_shape=jax.ShapeDtypeStruct((NSEG, 1), jnp.float32),
    )(partials, W1, b1.reshape(1, D), W2.reshape(1, D), b2.reshape(1, 1))


def kernel(scaler, vector, batch_index, W1, b1, W2, b2):
    zeros = jnp.zeros((NSEG, D), jnp.float32)
    partials = _sc_segment_sum(scaler, batch_index, zeros)
    return _mlp(partials, W1, b1, W2, b2)


# speedup vs baseline: 8.6227x; 1.8573x over previous
import functools

import jax
import jax.numpy as jnp
from jax import lax
from jax.experimental import pallas as pl
from jax.experimental.pallas import tpu as pltpu
from jax.experimental.pallas import tpu_sc as plsc

NSEG = 1024
NROWS = 320000
D = 128
NW = 32
UNITS = NROWS // 128
WU = 80
LAST_WU = UNITS - 31 * WU
CHUNK_U = 2


def _sc_segment_sum(scaler, batch_index2d, zeros):
    mesh = plsc.VectorSubcoreMesh(core_axis_name="c", subcore_axis_name="s")

    @functools.partial(
        pl.kernel,
        mesh=mesh,
        out_type=jax.ShapeDtypeStruct((2 * NSEG, D), jnp.float32),
        scratch_types=[
            pltpu.VMEM_SHARED((NSEG, D), jnp.float32),
            pltpu.VMEM((CHUNK_U * 128, D), jnp.float32),
            pltpu.VMEM((CHUNK_U * 128, D), jnp.float32),
            pltpu.VMEM((WU, 128), jnp.int32),
            pltpu.VMEM((NSEG // 16, D), jnp.float32),
            pltpu.SemaphoreType.DMA,
            pltpu.SemaphoreType.DMA,
        ],
    )
    def seg_sum(scaler_hbm, idx_hbm, zeros_hbm, out_hbm,
                acc, rows_a, rows_b, idx_all, bounce, sem_a, sem_b):
        c = lax.axis_index("c")
        s = lax.axis_index("s")
        wid = s * 2 + c

        seg0 = s * (NSEG // 16)
        pltpu.sync_copy(zeros_hbm.at[pl.ds(seg0, NSEG // 16)], bounce)
        pltpu.sync_copy(bounce, acc.at[pl.ds(seg0, NSEG // 16)])
        plsc.subcore_barrier()

        start_u = wid * WU
        nchunk = jnp.where(wid < NW - 1, WU // CHUNK_U, LAST_WU // CHUNK_U)

        pltpu.sync_copy(idx_hbm.at[pl.ds(start_u, WU)], idx_all)

        def load(chunk, buf, sem):
            base = (start_u + chunk * CHUNK_U) * 128
            pltpu.make_async_copy(
                scaler_hbm.at[pl.ds(base, CHUNK_U * 128)], buf, sem).start()

        def drain(buf, sem):
            pltpu.make_async_copy(
                scaler_hbm.at[pl.ds(0, CHUNK_U * 128)], buf, sem).wait()

        def scatter(chunk, buf):
            for j in range(CHUNK_U):
                pltpu.sync_copy(buf.at[pl.ds(j * 128, 128)],
                                acc.at[idx_all.at[chunk * CHUNK_U + j]],
                                add=True)

        load(0, rows_a, sem_a)
        load(1, rows_b, sem_b)

        def body(i, _):
            ca = 2 * i
            cb = 2 * i + 1
            drain(rows_a, sem_a)
            scatter(ca, rows_a)

            @pl.when(ca + 2 < nchunk)
            def _():
                load(ca + 2, rows_a, sem_a)

            drain(rows_b, sem_b)
            scatter(cb, rows_b)

            @pl.when(cb + 2 < nchunk)
            def _():
                load(cb + 2, rows_b, sem_b)
            return ()

        lax.fori_loop(0, nchunk // 2, body, ())
        plsc.subcore_barrier()

        pltpu.sync_copy(acc.at[pl.ds(seg0, NSEG // 16)], bounce)
        pltpu.sync_copy(bounce, out_hbm.at[pl.ds(c * NSEG + seg0, NSEG // 16)])

    return seg_sum(scaler, batch_index2d, zeros)


def _mlp_body(p_ref, w1_ref, b1_ref, w2_ref, b2_ref, o_ref):
    x = p_ref[0:NSEG, :] + p_ref[NSEG:2 * NSEG, :]
    h = jnp.dot(x, w1_ref[...], preferred_element_type=jnp.float32) + b1_ref[...]
    h = jnp.maximum(h, 0.0)
    o = jnp.sum(h * w2_ref[...], axis=1, keepdims=True) + b2_ref[0, 0]
    o_ref[...] = o


def _mlp(partials, W1, b1, W2, b2):
    return pl.pallas_call(
        _mlp_body,
        out_shape=jax.ShapeDtypeStruct((NSEG, 1), jnp.float32),
    )(partials, W1, b1.reshape(1, D), W2.reshape(1, D), b2.reshape(1, 1))


def kernel(scaler, vector, batch_index, W1, b1, W2, b2):
    zeros = jnp.zeros((NSEG, D), jnp.float32)
    idx2d = jnp.pad(batch_index.reshape(UNITS, 128), ((0, NW * WU - UNITS), (0, 0)))
    partials = _sc_segment_sum(scaler, idx2d, zeros)
    return _mlp(partials, W1, b1, W2, b2)


# speedup vs baseline: 9.3811x; 1.0880x over previous
import functools

import jax
import jax.numpy as jnp
from jax import lax
from jax.experimental import pallas as pl
from jax.experimental.pallas import tpu as pltpu
from jax.experimental.pallas import tpu_sc as plsc

NSEG = 1024
NROWS = 320000
D = 128
NW = 32
UNITS = NROWS // 128
WU = 80
LAST_WU = UNITS - 31 * WU
NBUF = 5
LOOKAHEAD = 3
SC_LAG = 2


def _sc_segment_sum(scaler, batch_index2d, zeros):
    mesh = plsc.VectorSubcoreMesh(core_axis_name="c", subcore_axis_name="s")

    @functools.partial(
        pl.kernel,
        mesh=mesh,
        out_type=jax.ShapeDtypeStruct((2 * NSEG, D), jnp.float32),
        scratch_types=[
            pltpu.VMEM_SHARED((NSEG, D), jnp.float32),
            pltpu.VMEM((NBUF * 128, D), jnp.float32),
            pltpu.VMEM((WU, 128), jnp.int32),
            pltpu.VMEM((NSEG // 16, D), jnp.float32),
            pltpu.SemaphoreType.DMA,
            pltpu.SemaphoreType.DMA,
            pltpu.SemaphoreType.DMA,
            pltpu.SemaphoreType.DMA,
            pltpu.SemaphoreType.DMA,
            pltpu.SemaphoreType.DMA,
            pltpu.SemaphoreType.DMA,
            pltpu.SemaphoreType.DMA,
            pltpu.SemaphoreType.DMA,
            pltpu.SemaphoreType.DMA,
        ],
    )
    def seg_sum(scaler_hbm, idx_hbm, zeros_hbm, out_hbm,
                acc, rows, idx_all, bounce, *sems):
        ld_sems = sems[:NBUF]
        sc_sems = sems[NBUF:]
        c = lax.axis_index("c")
        s = lax.axis_index("s")
        wid = s * 2 + c

        seg0 = s * (NSEG // 16)
        pltpu.sync_copy(zeros_hbm.at[pl.ds(seg0, NSEG // 16)], bounce)
        pltpu.sync_copy(bounce, acc.at[pl.ds(seg0, NSEG // 16)])
        plsc.subcore_barrier()

        start_u = wid * WU
        nchunk = jnp.where(wid < NW - 1, WU, LAST_WU)

        pltpu.sync_copy(idx_hbm.at[pl.ds(start_u, WU)], idx_all)

        def buf(b):
            return rows.at[pl.ds(b * 128, 128)]

        def load(chunk, b):
            base = (start_u + chunk) * 128
            pltpu.make_async_copy(
                scaler_hbm.at[pl.ds(base, 128)], buf(b), ld_sems[b]).start()

        def drain_ld(b):
            pltpu.make_async_copy(
                scaler_hbm.at[pl.ds(0, 128)], buf(b), ld_sems[b]).wait()

        def scatter(chunk, b):
            pltpu.make_async_copy(
                buf(b), acc.at[idx_all.at[chunk]], sc_sems[b]).start(add=True)

        def drain_sc(b):
            pltpu.make_async_copy(
                buf(b), acc.at[idx_all.at[0]], sc_sems[b]).wait()

        for p in range(LOOKAHEAD):
            load(p, p)

        def body(i, _):
            for k in range(NBUF):
                chunk = i * NBUF + k

                @pl.when(chunk >= SC_LAG)
                def _():
                    drain_sc((k - SC_LAG) % NBUF)

                drain_ld(k)
                scatter(chunk, k)

                @pl.when(chunk + LOOKAHEAD < nchunk)
                def _():
                    load(chunk + LOOKAHEAD, (k + LOOKAHEAD) % NBUF)
            return ()

        lax.fori_loop(0, nchunk // NBUF, body, ())
        for t in range(SC_LAG):
            drain_sc((NBUF - SC_LAG + t) % NBUF)
        plsc.subcore_barrier()

        pltpu.sync_copy(acc.at[pl.ds(seg0, NSEG // 16)], bounce)
        pltpu.sync_copy(bounce, out_hbm.at[pl.ds(c * NSEG + seg0, NSEG // 16)])

    return seg_sum(scaler, batch_index2d, zeros)


def _mlp_body(p_ref, w1_ref, b1_ref, w2_ref, b2_ref, o_ref):
    x = p_ref[0:NSEG, :] + p_ref[NSEG:2 * NSEG, :]
    h = jnp.dot(x, w1_ref[...], preferred_element_type=jnp.float32) + b1_ref[...]
    h = jnp.maximum(h, 0.0)
    o = jnp.sum(h * w2_ref[...], axis=1, keepdims=True) + b2_ref[0, 0]
    o_ref[...] = o


def _mlp(partials, W1, b1, W2, b2):
    return pl.pallas_call(
        _mlp_body,
        out_shape=jax.ShapeDtypeStruct((NSEG, 1), jnp.float32),
    )(partials, W1, b1.reshape(1, D), W2.reshape(1, D), b2.reshape(1, 1))


def kernel(scaler, vector, batch_index, W1, b1, W2, b2):
    zeros = jnp.zeros((NSEG, D), jnp.float32)
    idx2d = jnp.pad(batch_index.reshape(UNITS, 128), ((0, NW * WU - UNITS), (0, 0)))
    partials = _sc_segment_sum(scaler, idx2d, zeros)
    return _mlp(partials, W1, b1, W2, b2)
